# baseline (speedup 1.0000x reference)
import jax
import jax.numpy as jnp
from jax import lax
from jax.experimental import pallas as pl
from jax.experimental.pallas import tpu as pltpu
from jax.experimental.pallas import tpu_sc as plsc

N = 10000
NP = 10240
E = 320000
D = 128
NC = 2
NS = 16
L = 16
K = 80
NB = 128
EPAD = NC * NS * NB * K
EPT = EPAD // (NC * NS)
NS4 = 4
QUADS = NB // NS4
RPT = NP // NS
ZR = 32


def _body(cols_h, rows_h, vals_h, b_h, out0_h, out1_h,
          cidx0, cidx1, cidx2, cidx3, ridx0, ridx1, ridx2, ridx3,
          rsh0, rsh1, rsh2, rsh3, vbuf0, vbuf1, vbuf2, vbuf3,
          gbuf0, gbuf1, gbuf2, gbuf3, zbuf, acc,
          semg0, semg1, semg2, semg3, sems0, sems1, sems2, sems3,
          semi0, semi1, semi2, semi3, zsem):
    cidx = [cidx0, cidx1, cidx2, cidx3]
    ridx = [ridx0, ridx1, ridx2, ridx3]
    rsh = [rsh0, rsh1, rsh2, rsh3]
    vbuf = [vbuf0, vbuf1, vbuf2, vbuf3]
    gbuf = [gbuf0, gbuf1, gbuf2, gbuf3]
    semg = [semg0, semg1, semg2, semg3]
    sems = [sems0, sems1, sems2, sems3]
    semi = [semi0, semi1, semi2, semi3]

    c = lax.axis_index("c")
    s = lax.axis_index("s")
    ebase = (c * NS + s) * EPT
    rbase = s * RPT

    def stage(j, m):
        esl = pl.ds(ebase + j * K, K)
        pltpu.async_copy(cols_h.at[esl], cidx[m], semi[m])
        pltpu.async_copy(rows_h.at[esl], ridx[m], semi[m])
        pltpu.async_copy(vals_h.at[esl], vbuf[m], semi[m])

    def wait_stage(m):
        esl = pl.ds(0, K)
        pltpu.make_async_copy(cols_h.at[esl], cidx[m], semi[m]).wait()
        pltpu.make_async_copy(rows_h.at[esl], ridx[m], semi[m]).wait()
        pltpu.make_async_copy(vals_h.at[esl], vbuf[m], semi[m]).wait()

    def fire_gather(m):
        pltpu.async_copy(b_h.at[cidx[m]], gbuf[m], semg[m])

    def wait_gather(m):
        pltpu.make_async_copy(b_h.at[cidx[m]], gbuf[m], semg[m]).wait()

    def snapshot(m):
        vvecs = []
        for t in range(K // L):
            sl = pl.ds(t * L, L)
            rsh[m][sl] = ridx[m][sl]
            vvecs.append(vbuf[m][sl])
        return vvecs

    def scale(m, vvecs):
        for t in range(K // L):
            for u in range(L):
                i = t * L + u
                v = vvecs[t][u]
                for q in range(D // L):
                    sl = pl.ds(q * L, L)
                    gbuf[m][i, sl] = gbuf[m][i, sl] * v

    def fire_scatter(m):
        pltpu.async_copy(gbuf[m], acc.at[rsh[m]], sems[m], add=True)

    def wait_scatter(m):
        pltpu.make_async_copy(gbuf[m], acc.at[rsh[m]], sems[m]).wait()

    zero = jnp.zeros((L,), jnp.float32)

    def zrow(i, carry):
        for q in range(D // L):
            zbuf[i, pl.ds(q * L, L)] = zero
        return carry

    lax.fori_loop(0, ZR, zrow, 0)
    for z in range(RPT // ZR):
        pltpu.async_copy(zbuf, acc.at[pl.ds(rbase + z * ZR, ZR)], zsem)
    for z in range(RPT // ZR):
        pltpu.make_async_copy(zbuf, acc.at[pl.ds(rbase + z * ZR, ZR)],
                              zsem).wait()
    plsc.subcore_barrier()

    stage(0, 0)
    stage(1, 1)
    stage(2, 2)
    wait_stage(0)
    fire_gather(0)
    wait_stage(1)
    fire_gather(1)

    def quad(jo, carry):
        for bi in range(NS4):
            j = NS4 * jo + bi
            m2 = (bi + 2) % NS4
            wait_gather(bi)
            vvecs = snapshot(bi)
            if bi == 0:
                stage(j + 3, (bi + 3) % NS4)
            else:
                @pl.when(jo < QUADS - 1)
                def _():
                    stage(j + 3, (bi + 3) % NS4)
            scale(bi, vvecs)
            fire_scatter(bi)
            if bi >= 2:
                wait_scatter(m2)

                @pl.when(jo < QUADS - 1)
                def _():
                    wait_stage(m2)
                    fire_gather(m2)
            else:
                @pl.when(jo >= 1)
                def _():
                    wait_scatter(m2)

                wait_stage(m2)
                fire_gather(m2)
        return carry

    lax.fori_loop(0, QUADS, quad, 0)

    wait_scatter(2)
    wait_scatter(3)
    plsc.subcore_barrier()

    osl = pl.ds(rbase, RPT)

    @pl.when(c == 0)
    def _():
        pltpu.sync_copy(acc.at[osl], out0_h.at[osl])

    @pl.when(c == 1)
    def _():
        pltpu.sync_copy(acc.at[osl], out1_h.at[osl])


_spmm = pl.kernel(
    _body,
    out_type=(jax.ShapeDtypeStruct((NP, D), jnp.float32),
              jax.ShapeDtypeStruct((NP, D), jnp.float32)),
    mesh=plsc.VectorSubcoreMesh(
        core_axis_name="c", subcore_axis_name="s",
        num_cores=NC, num_subcores=NS),
    scratch_types=(
        [pltpu.VMEM((K,), jnp.int32) for _ in range(4)]
        + [pltpu.VMEM((K,), jnp.int32) for _ in range(4)]
        + [pltpu.VMEM((K,), jnp.int32) for _ in range(4)]
        + [pltpu.VMEM((K,), jnp.float32) for _ in range(4)]
        + [pltpu.VMEM((K, D), jnp.float32) for _ in range(4)]
        + [pltpu.VMEM((ZR, D), jnp.float32)]
        + [pltpu.VMEM_SHARED((NP, D), jnp.float32)]
        + [pltpu.SemaphoreType.DMA for _ in range(13)]
    ),
)


def _add_body(a_ref, b_ref, o_ref):
    sl = pl.ds(0, N)
    o_ref[...] = a_ref[sl, :] + b_ref[sl, :]


_combine = pl.pallas_call(
    _add_body,
    out_shape=jax.ShapeDtypeStruct((N, D), jnp.float32),
)


def kernel(edge_index, values, B):
    pad = EPAD - E
    rows = jnp.pad(edge_index[0], (0, pad))
    cols = jnp.pad(edge_index[1], (0, pad))
    vals = jnp.pad(values, (0, pad))
    p0, p1 = _spmm(cols, rows, vals, B)
    return _combine(p0, p1)


# speedup vs baseline: 2.8487x; 2.8487x over previous
import jax
import jax.numpy as jnp
from jax import lax
from jax.experimental import pallas as pl
from jax.experimental.pallas import tpu as pltpu
from jax.experimental.pallas import tpu_sc as plsc

N = 10000
NP = 10240
E = 320000
D = 128
NC = 2
NS = 16
L = 16
EPT = E // (NC * NS)
K = 80
NB = EPT // K
PAIRS = NB // 2
RPT = NP // NS
ZR = 32


def _body(cols_h, rows_h, vals_h, b_h, out0_h, out1_h,
          cidx0, cidx1, ridx0, ridx1, rsh0, rsh1, vbuf0, vbuf1,
          gbuf0, gbuf1, sbuf0, sbuf1, zbuf, acc,
          semg0, semg1, sems0, sems1, semi0, semi1, zsem):
    cidx = [cidx0, cidx1]
    ridx = [ridx0, ridx1]
    rsh = [rsh0, rsh1]
    vbuf = [vbuf0, vbuf1]
    gbuf = [gbuf0, gbuf1]
    sbuf = [sbuf0, sbuf1]
    semg = [semg0, semg1]
    sems = [sems0, sems1]
    semi = [semi0, semi1]

    c = lax.axis_index("c")
    s = lax.axis_index("s")
    ebase = (c * NS + s) * EPT
    rbase = s * RPT

    def stage(j, b):
        esl = pl.ds(ebase + j * K, K)
        pltpu.async_copy(cols_h.at[esl], cidx[b], semi[b])
        pltpu.async_copy(rows_h.at[esl], ridx[b], semi[b])
        pltpu.async_copy(vals_h.at[esl], vbuf[b], semi[b])

    def wait_stage(b):
        esl = pl.ds(0, K)
        pltpu.make_async_copy(cols_h.at[esl], cidx[b], semi[b]).wait()
        pltpu.make_async_copy(rows_h.at[esl], ridx[b], semi[b]).wait()
        pltpu.make_async_copy(vals_h.at[esl], vbuf[b], semi[b]).wait()

    def fire_gather(b):
        pltpu.async_copy(b_h.at[cidx[b]], gbuf[b], semg[b])

    def wait_gather(b):
        pltpu.make_async_copy(b_h.at[cidx[b]], gbuf[b], semg[b]).wait()

    def snapshot(b):
        vvecs = []
        for t in range(K // L):
            sl = pl.ds(t * L, L)
            rsh[b][sl] = ridx[b][sl]
            vvecs.append(vbuf[b][sl])
        return vvecs

    def scale(b, vvecs):
        for t in range(K // L):
            for u in range(L):
                i = t * L + u
                v = vvecs[t][u]
                for q in range(D // L):
                    sl = pl.ds(q * L, L)
                    sbuf[b][i, sl] = gbuf[b][i, sl] * v

    def fire_scatter(b):
        pltpu.async_copy(sbuf[b], acc.at[rsh[b]], sems[b], add=True)

    def wait_scatter(b):
        pltpu.make_async_copy(sbuf[b], acc.at[rsh[b]], sems[b]).wait()

    zero = jnp.zeros((L,), jnp.float32)

    def zrow(i, carry):
        for q in range(D // L):
            zbuf[i, pl.ds(q * L, L)] = zero
        return carry

    lax.fori_loop(0, ZR, zrow, 0)
    for z in range(RPT // ZR):
        pltpu.async_copy(zbuf, acc.at[pl.ds(rbase + z * ZR, ZR)], zsem)
    for z in range(RPT // ZR):
        pltpu.make_async_copy(zbuf, acc.at[pl.ds(rbase + z * ZR, ZR)],
                              zsem).wait()
    plsc.subcore_barrier()

    stage(0, 0)
    wait_stage(0)
    fire_gather(0)
    stage(1, 1)

    def pair(j2, carry):
        for b in range(2):
            j = 2 * j2 + b
            wait_gather(b)
            wait_stage(1 - b)
            fire_gather(1 - b)

            @pl.when(j2 >= 1)
            def _():
                wait_scatter(b)

            vvecs = snapshot(b)
            if b == 0:
                stage(j + 2, 0)
            else:
                @pl.when(j2 < PAIRS - 1)
                def _():
                    stage(j + 2, 1)
            scale(b, vvecs)
            fire_scatter(b)
        return carry

    lax.fori_loop(0, PAIRS, pair, 0)

    wait_gather(0)
    wait_scatter(0)
    vvecs = snapshot(0)
    scale(0, vvecs)
    fire_scatter(0)
    wait_scatter(0)
    wait_scatter(1)
    plsc.subcore_barrier()

    osl = pl.ds(rbase, RPT)

    @pl.when(c == 0)
    def _():
        pltpu.sync_copy(acc.at[osl], out0_h.at[osl])

    @pl.when(c == 1)
    def _():
        pltpu.sync_copy(acc.at[osl], out1_h.at[osl])


_spmm = pl.kernel(
    _body,
    out_type=(jax.ShapeDtypeStruct((NP, D), jnp.float32),
              jax.ShapeDtypeStruct((NP, D), jnp.float32)),
    mesh=plsc.VectorSubcoreMesh(
        core_axis_name="c", subcore_axis_name="s",
        num_cores=NC, num_subcores=NS),
    scratch_types=[
        pltpu.VMEM((K,), jnp.int32),
        pltpu.VMEM((K,), jnp.int32),
        pltpu.VMEM((K,), jnp.int32),
        pltpu.VMEM((K,), jnp.int32),
        pltpu.VMEM((K,), jnp.int32),
        pltpu.VMEM((K,), jnp.int32),
        pltpu.VMEM((K,), jnp.float32),
        pltpu.VMEM((K,), jnp.float32),
        pltpu.VMEM((K, D), jnp.float32),
        pltpu.VMEM((K, D), jnp.float32),
        pltpu.VMEM((K, D), jnp.float32),
        pltpu.VMEM((K, D), jnp.float32),
        pltpu.VMEM((ZR, D), jnp.float32),
        pltpu.VMEM_SHARED((NP, D), jnp.float32),
        pltpu.SemaphoreType.DMA,
        pltpu.SemaphoreType.DMA,
        pltpu.SemaphoreType.DMA,
        pltpu.SemaphoreType.DMA,
        pltpu.SemaphoreType.DMA,
        pltpu.SemaphoreType.DMA,
        pltpu.SemaphoreType.DMA,
    ],
)


def _add_body(a_ref, b_ref, o_ref):
    sl = pl.ds(0, N)
    o_ref[...] = a_ref[sl, :] + b_ref[sl, :]


_combine = pl.pallas_call(
    _add_body,
    out_shape=jax.ShapeDtypeStruct((N, D), jnp.float32),
)


def kernel(edge_index, values, B):
    rows = edge_index[0]
    cols = edge_index[1]
    p0, p1 = _spmm(cols, rows, values, B)
    return _combine(p0, p1)
